# Initial kernel scaffold; baseline (speedup 1.0000x reference)
import jax
import jax.numpy as jnp
from jax.experimental import pallas as pl


def kernel(pheatmap, pwh, pxy_offset, pkeypoint_offset):
    raise NotImplementedError("write your pallas kernel here")


# speedup vs baseline: 1.6363x; 1.6363x over previous
import functools
import jax
import jax.numpy as jnp
from jax import lax
from jax.experimental import pallas as pl
from jax.experimental.pallas import tpu as pltpu

_THRESHOLD = 0.18
_TOPK = 100
_B, _C, _H, _W = 8, 80, 128, 128
_NEG = -2147483648


def _to_key(x):
    b = lax.bitcast_convert_type(x, jnp.int32)
    return jnp.where(b >= 0, b, b ^ jnp.int32(0x7FFFFFFF))


def _from_key(k):
    b = jnp.where(k >= 0, k, k ^ jnp.int32(0x7FFFFFFF))
    return lax.bitcast_convert_type(b, jnp.float32)


def _nms_kernel(x_ref, out_ref, acc_ref):
    c = pl.program_id(1)
    x = x_ref[0, 0]
    lane = lax.broadcasted_iota(jnp.int32, (_H, _W), 1)
    sub = lax.broadcasted_iota(jnp.int32, (_H, _W), 0)
    ninf = jnp.float32(-jnp.inf)
    sr = jnp.where(lane == 0, ninf, pltpu.roll(x, 1, 1))
    sl = jnp.where(lane == _W - 1, ninf, pltpu.roll(x, _W - 1, 1))
    m1 = jnp.maximum(jnp.maximum(x, sr), sl)
    su = jnp.where(sub == 0, ninf, pltpu.roll(m1, 1, 0))
    sd = jnp.where(sub == _H - 1, ninf, pltpu.roll(m1, _H - 1, 0))
    m2 = jnp.maximum(jnp.maximum(m1, su), sd)
    cand = jnp.where(m2 == x, x, jnp.float32(0.0))

    @pl.when(c == 0)
    def _():
        acc_ref[...] = cand

    @pl.when(c > 0)
    def _():
        acc_ref[...] = jnp.maximum(acc_ref[...], cand)

    @pl.when(c == _C - 1)
    def _():
        acc = acc_ref[...]
        masked = jnp.where(acc > _THRESHOLD, acc, ninf)
        out_ref[0] = _to_key(masked)


def _topk_kernel(keys_ref, pwh_ref, pxy_ref, ids_ref, boxes_ref, sc_ref,
                 skeys_ref):
    skeys_ref[...] = keys_ref[...]
    tilemax0 = jnp.max(keys_ref[...], axis=(1, 2)).reshape(1, 128)
    lane1 = lax.broadcasted_iota(jnp.int32, (1, 128), 1)
    pos = (lax.broadcasted_iota(jnp.int32, (8, 128), 0) * 128
           + lax.broadcasted_iota(jnp.int32, (8, 128), 1))
    zf = jnp.zeros((1, 128), jnp.float32)
    zi = jnp.zeros((1, 128), jnp.int32)

    def body(i, carry):
        tilemax, ids, sc, bl, bt, br, bb = carry
        m = jnp.max(tilemax)
        tstar = jnp.min(jnp.where(tilemax == m, lane1, jnp.int32(128)))
        tile = skeys_ref[tstar]
        pstar = jnp.min(jnp.where(tile == m, pos, jnp.int32(1024)))
        newtile = jnp.where(pos == pstar, jnp.int32(_NEG), tile)
        skeys_ref[tstar] = newtile
        tilemax = jnp.where(lane1 == tstar, jnp.max(newtile), tilemax)

        flat = tstar * 1024 + pstar
        b = flat // (_H * _W)
        rem = flat - b * (_H * _W)
        y = rem // _W
        xcol = rem - y * _W

        sval = _from_key(m)
        sel = lane1 == xcol
        rw = pwh_ref[pl.ds(b * 2 * _H + y, 1), :]
        rh = pwh_ref[pl.ds((b * 2 + 1) * _H + y, 1), :]
        rx = pxy_ref[pl.ds(b * 2 * _H + y, 1), :]
        ry = pxy_ref[pl.ds((b * 2 + 1) * _H + y, 1), :]
        w = jnp.sum(jnp.where(sel, rw, zf))
        h = jnp.sum(jnp.where(sel, rh, zf))
        ox = jnp.sum(jnp.where(sel, rx, zf))
        oy = jnp.sum(jnp.where(sel, ry, zf))
        xc = (ox + xcol.astype(jnp.float32)) / jnp.float32(_W)
        yc = (oy + y.astype(jnp.float32)) / jnp.float32(_H)

        oh = lane1 == i
        ids = jnp.where(oh, b, ids)
        sc = jnp.where(oh, sval, sc)
        bl = jnp.where(oh, xc - w * 0.5, bl)
        bt = jnp.where(oh, yc - h * 0.5, bt)
        br = jnp.where(oh, xc + w * 0.5, br)
        bb = jnp.where(oh, yc + h * 0.5, bb)
        return tilemax, ids, sc, bl, bt, br, bb

    tilemax, ids, sc, bl, bt, br, bb = lax.fori_loop(
        0, _TOPK, body, (tilemax0, zi, zf, zf, zf, zf, zf))
    ids_ref[...] = ids
    sc_ref[...] = sc
    boxes_ref[...] = jnp.concatenate([bl, bt, br, bb], axis=0)


def kernel(pheatmap, pwh, pxy_offset, pkeypoint_offset):
    del pkeypoint_offset
    keys = pl.pallas_call(
        _nms_kernel,
        grid=(_B, _C),
        in_specs=[pl.BlockSpec((1, 1, _H, _W), lambda b, c: (b, c, 0, 0))],
        out_specs=pl.BlockSpec((1, _H, _W), lambda b, c: (b, 0, 0)),
        out_shape=jax.ShapeDtypeStruct((_B, _H, _W), jnp.int32),
        scratch_shapes=[pltpu.VMEM((_H, _W), jnp.float32)],
    )(pheatmap)

    keys3 = keys.reshape(_B * _H // 8, 8, _W)
    pwh2 = pwh.reshape(_B * 2 * _H, _W)
    pxy2 = pxy_offset.reshape(_B * 2 * _H, _W)

    ids_p, boxes_p, sc_p = pl.pallas_call(
        _topk_kernel,
        out_shape=[jax.ShapeDtypeStruct((1, 128), jnp.int32),
                   jax.ShapeDtypeStruct((4, 128), jnp.float32),
                   jax.ShapeDtypeStruct((1, 128), jnp.float32)],
        scratch_shapes=[pltpu.VMEM((_B * _H // 8, 8, _W), jnp.int32)],
    )(keys3, pwh2, pxy2)

    ids = ids_p[0, :_TOPK]
    scores = sc_p[0, :_TOPK]
    boxes = boxes_p[:, :_TOPK].T
    return ids, boxes, scores, scores


# speedup vs baseline: 4.5072x; 2.7544x over previous
import functools
import jax
import jax.numpy as jnp
from jax import lax
from jax.experimental import pallas as pl
from jax.experimental.pallas import tpu as pltpu

_THRESHOLD = 0.18
_TOPK = 100
_B, _C, _H, _W = 8, 80, 128, 128
_NEG = -2147483648


def _to_key(x):
    b = lax.bitcast_convert_type(x, jnp.int32)
    return jnp.where(b >= 0, b, b ^ jnp.int32(0x7FFFFFFF))


def _from_key(k):
    b = jnp.where(k >= 0, k, k ^ jnp.int32(0x7FFFFFFF))
    return lax.bitcast_convert_type(b, jnp.float32)


_CBLK = 8
_NCB = _C // _CBLK


def _nms_kernel(x_ref, out_ref, acc_ref):
    c = pl.program_id(1)
    x = x_ref[0]
    lane = lax.broadcasted_iota(jnp.int32, (_CBLK, _H, _W), 2)
    sub = lax.broadcasted_iota(jnp.int32, (_CBLK, _H, _W), 1)
    ninf = jnp.float32(-jnp.inf)
    sr = jnp.where(lane == 0, ninf, pltpu.roll(x, 1, 2))
    sl = jnp.where(lane == _W - 1, ninf, pltpu.roll(x, _W - 1, 2))
    m1 = jnp.maximum(jnp.maximum(x, sr), sl)
    su = jnp.where(sub == 0, ninf, pltpu.roll(m1, 1, 1))
    sd = jnp.where(sub == _H - 1, ninf, pltpu.roll(m1, _H - 1, 1))
    m2 = jnp.maximum(jnp.maximum(m1, su), sd)
    cand = jnp.max(jnp.where(m2 == x, x, jnp.float32(0.0)), axis=0)

    @pl.when(c == 0)
    def _():
        acc_ref[...] = cand

    @pl.when(c > 0)
    def _():
        acc_ref[...] = jnp.maximum(acc_ref[...], cand)

    @pl.when(c == _NCB - 1)
    def _():
        acc = jnp.maximum(acc_ref[...], cand)
        masked = jnp.where(acc > _THRESHOLD, acc, ninf)
        out_ref[0] = _to_key(masked)


def _topk_kernel(keys_ref, pwh_ref, pxy_ref, ids_ref, boxes_ref, sc_ref,
                 skeys_ref):
    skeys_ref[...] = keys_ref[...]
    tilemax0 = jnp.max(keys_ref[...], axis=(1, 2)).reshape(1, 128)
    lane1 = lax.broadcasted_iota(jnp.int32, (1, 128), 1)
    pos = (lax.broadcasted_iota(jnp.int32, (8, 128), 0) * 128
           + lax.broadcasted_iota(jnp.int32, (8, 128), 1))
    zf = jnp.zeros((1, 128), jnp.float32)
    zi = jnp.zeros((1, 128), jnp.int32)

    def body(i, carry):
        tilemax, ids, sc, bl, bt, br, bb = carry
        m = jnp.max(tilemax)
        tstar = jnp.min(jnp.where(tilemax == m, lane1, jnp.int32(128)))
        tile = skeys_ref[tstar]
        pstar = jnp.min(jnp.where(tile == m, pos, jnp.int32(1024)))
        newtile = jnp.where(pos == pstar, jnp.int32(_NEG), tile)
        skeys_ref[tstar] = newtile
        tilemax = jnp.where(lane1 == tstar, jnp.max(newtile), tilemax)

        flat = tstar * 1024 + pstar
        b = flat // (_H * _W)
        rem = flat - b * (_H * _W)
        y = rem // _W
        xcol = rem - y * _W

        sval = _from_key(m)
        sel = lane1 == xcol
        rw = pwh_ref[pl.ds(b * 2 * _H + y, 1), :]
        rh = pwh_ref[pl.ds((b * 2 + 1) * _H + y, 1), :]
        rx = pxy_ref[pl.ds(b * 2 * _H + y, 1), :]
        ry = pxy_ref[pl.ds((b * 2 + 1) * _H + y, 1), :]
        w = jnp.sum(jnp.where(sel, rw, zf))
        h = jnp.sum(jnp.where(sel, rh, zf))
        ox = jnp.sum(jnp.where(sel, rx, zf))
        oy = jnp.sum(jnp.where(sel, ry, zf))
        xc = (ox + xcol.astype(jnp.float32)) / jnp.float32(_W)
        yc = (oy + y.astype(jnp.float32)) / jnp.float32(_H)

        oh = lane1 == i
        ids = jnp.where(oh, b, ids)
        sc = jnp.where(oh, sval, sc)
        bl = jnp.where(oh, xc - w * 0.5, bl)
        bt = jnp.where(oh, yc - h * 0.5, bt)
        br = jnp.where(oh, xc + w * 0.5, br)
        bb = jnp.where(oh, yc + h * 0.5, bb)
        return tilemax, ids, sc, bl, bt, br, bb

    tilemax, ids, sc, bl, bt, br, bb = lax.fori_loop(
        0, _TOPK, body, (tilemax0, zi, zf, zf, zf, zf, zf))
    ids_ref[...] = ids
    sc_ref[...] = sc
    boxes_ref[...] = jnp.concatenate([bl, bt, br, bb], axis=0)


def kernel(pheatmap, pwh, pxy_offset, pkeypoint_offset):
    del pkeypoint_offset
    keys = pl.pallas_call(
        _nms_kernel,
        grid=(_B, _NCB),
        in_specs=[pl.BlockSpec((1, _CBLK, _H, _W), lambda b, c: (b, c, 0, 0))],
        out_specs=pl.BlockSpec((1, _H, _W), lambda b, c: (b, 0, 0)),
        out_shape=jax.ShapeDtypeStruct((_B, _H, _W), jnp.int32),
        scratch_shapes=[pltpu.VMEM((_H, _W), jnp.float32)],
    )(pheatmap)

    keys3 = keys.reshape(_B * _H // 8, 8, _W)
    pwh2 = pwh.reshape(_B * 2 * _H, _W)
    pxy2 = pxy_offset.reshape(_B * 2 * _H, _W)

    ids_p, boxes_p, sc_p = pl.pallas_call(
        _topk_kernel,
        out_shape=[jax.ShapeDtypeStruct((1, 128), jnp.int32),
                   jax.ShapeDtypeStruct((4, 128), jnp.float32),
                   jax.ShapeDtypeStruct((1, 128), jnp.float32)],
        scratch_shapes=[pltpu.VMEM((_B * _H // 8, 8, _W), jnp.int32)],
    )(keys3, pwh2, pxy2)

    ids = ids_p[0, :_TOPK]
    scores = sc_p[0, :_TOPK]
    boxes = boxes_p[:, :_TOPK].T
    return ids, boxes, scores, scores


# speedup vs baseline: 5.1174x; 1.1354x over previous
import functools
import jax
import jax.numpy as jnp
from jax import lax
from jax.experimental import pallas as pl
from jax.experimental.pallas import tpu as pltpu
from jax.experimental.pallas import tpu_sc as plsc

_THRESHOLD = 0.18
_TOPK = 100
_B, _C, _H, _W = 8, 80, 128, 128
_MIN = -2147483648
_CBLK = 8
_NCB = _C // _CBLK
_NW = 16
_CHUNK = (_B * _H * _W) // _NW
_NG = _CHUNK // 16
_NJ = _NG // 16


def _to_key(x):
    b = lax.bitcast_convert_type(x, jnp.int32)
    return jnp.where(b >= 0, b, b ^ jnp.int32(0x7FFFFFFF))


def _nms_kernel(x_ref, out_ref, acc_ref):
    c = pl.program_id(1)
    x = x_ref[0]
    lane = lax.broadcasted_iota(jnp.int32, (_CBLK, _H, _W), 2)
    sub = lax.broadcasted_iota(jnp.int32, (_CBLK, _H, _W), 1)
    ninf = jnp.float32(-jnp.inf)
    sr = jnp.where(lane == 0, ninf, pltpu.roll(x, 1, 2))
    sl = jnp.where(lane == _W - 1, ninf, pltpu.roll(x, _W - 1, 2))
    m1 = jnp.maximum(jnp.maximum(x, sr), sl)
    su = jnp.where(sub == 0, ninf, pltpu.roll(m1, 1, 1))
    sd = jnp.where(sub == _H - 1, ninf, pltpu.roll(m1, _H - 1, 1))
    m2 = jnp.maximum(jnp.maximum(m1, su), sd)
    cand = jnp.max(jnp.where(m2 == x, x, jnp.float32(0.0)), axis=0)

    @pl.when(c == 0)
    def _():
        acc_ref[...] = cand

    @pl.when(c > 0)
    def _():
        acc_ref[...] = jnp.maximum(acc_ref[...], cand)

    @pl.when(c == _NCB - 1)
    def _():
        acc = jnp.maximum(acc_ref[...], cand)
        masked = jnp.where(acc > _THRESHOLD, acc, ninf)
        out_ref[0] = _to_key(masked)


def _iota16():
    return lax.iota(jnp.int32, 16)


def _rmw_lane(ref, pos, val):
    base = (pos // 16) * 16
    lane = pos - base
    v = ref[pl.ds(base, 16)]
    ref[pl.ds(base, 16)] = jnp.where(_iota16() == lane, val, v)


def _sc_body(keys_hbm, ids_hbm, pix_hbm, sc_hbm,
             kv, cmx, gmax, g2, lk, li, shk, shi, mk, mi,
             selk, seli, idsb, scb):
    cid = lax.axis_index("c")
    wid = lax.axis_index("s")
    active = cid == 0
    w0 = jnp.logical_and(active, wid == 0)
    it = _iota16()

    @pl.when(active)
    def _():
        pltpu.sync_copy(keys_hbm.at[pl.ds(wid * _CHUNK, _CHUNK)], kv)
        def p1(g, _):
            cmx[pl.ds(g * 16, 16)] = plsc.cummax(kv[pl.ds(g * 16, 16)])
            return 0
        lax.fori_loop(0, _NG, p1, 0)

        def p1b(j, _):
            gmax[pl.ds(j * 16, 16)] = plsc.load_gather(
                cmx, [j * 256 + it * 16 + 15])
            return 0
        lax.fori_loop(0, _NJ, p1b, 0)

        def p1c(j, _):
            cmx[pl.ds(j * 16, 16)] = plsc.cummax(gmax[pl.ds(j * 16, 16)])
            return 0
        lax.fori_loop(0, _NJ, p1c, 0)
        g2[pl.ds(0, 16)] = plsc.load_gather(cmx, [it * 16 + 15])
        g2[pl.ds(16, 16)] = plsc.load_gather(cmx, [256 + it * 16 + 15])

        def p2(r, _):
            va = g2[pl.ds(0, 16)]
            vb = g2[pl.ds(16, 16)]
            take = vb > va
            cv = jnp.where(take, vb, va)
            cj = jnp.where(take, it + 16, it)
            m = jnp.max(cv)
            jstar = jnp.min(jnp.where(cv == m, cj, jnp.int32(31)))
            gv = gmax[pl.ds(jstar * 16, 16)]
            gin = jnp.min(jnp.where(gv == m, it, jnp.int32(15)))
            gstar = jstar * 16 + gin
            kvv = kv[pl.ds(gstar * 16, 16)]
            lstar = jnp.min(jnp.where(kvv == m, it, jnp.int32(15)))
            lidx = gstar * 16 + lstar

            kvv2 = jnp.where(it == lstar, jnp.int32(_MIN), kvv)
            kv[pl.ds(gstar * 16, 16)] = kvv2
            gv2 = jnp.where(it == gin, jnp.max(kvv2), gv)
            gmax[pl.ds(jstar * 16, 16)] = gv2
            ng2 = jnp.max(gv2)
            base2 = (jstar // 16) * 16
            v2 = g2[pl.ds(base2, 16)]
            g2[pl.ds(base2, 16)] = jnp.where(it == jstar - base2, ng2, v2)

            _rmw_lane(lk, r, m)
            _rmw_lane(li, r, wid * _CHUNK + lidx)
            return 0
        lax.fori_loop(0, _TOPK, p2, 0)

        for t in range(_TOPK // 16, 8):
            padmask = (t * 16 + it) >= _TOPK
            lk[pl.ds(t * 16, 16)] = jnp.where(padmask, jnp.int32(_MIN),
                                              lk[pl.ds(t * 16, 16)])
            li[pl.ds(t * 16, 16)] = jnp.where(padmask, jnp.int32(0),
                                              li[pl.ds(t * 16, 16)])
        pltpu.sync_copy(lk, shk.at[pl.ds(wid * 128, 128)])
        pltpu.sync_copy(li, shi.at[pl.ds(wid * 128, 128)])

    plsc.subcore_barrier()

    @pl.when(w0)
    def _():
        pltpu.sync_copy(shk, mk)
        pltpu.sync_copy(shi, mi)
        heads0 = plsc.load_gather(mk, [it * 128])
        ptrs0 = jnp.zeros((16,), jnp.int32)

        def p4(i, carry):
            heads, ptrs = carry
            m = jnp.max(heads)
            wstar = jnp.min(jnp.where(heads == m, it, jnp.int32(15)))
            p = jnp.minimum(
                jnp.min(jnp.where(it == wstar, ptrs, jnp.int32(126))),
                jnp.int32(126))
            gidx = mi[pl.ds(wstar * 128 + p, 16)][0]
            nxt = mk[pl.ds(wstar * 128 + p + 1, 16)][0]
            heads = jnp.where(it == wstar, nxt, heads)
            ptrs = jnp.where(it == wstar, p + 1, ptrs)
            _rmw_lane(selk, i, m)
            _rmw_lane(seli, i, gidx)
            return heads, ptrs
        lax.fori_loop(0, _TOPK, p4, (heads0, ptrs0))

        for t in range(8):
            sl = pl.ds(t * 16, 16)
            iv = seli[sl]
            idsb[sl] = lax.shift_right_arithmetic(iv, 14)
            k = selk[sl]
            bits = jnp.where(k >= 0, k, k ^ jnp.int32(0x7FFFFFFF))
            scb[sl] = plsc.bitcast(bits, jnp.float32)

        pltpu.sync_copy(idsb, ids_hbm)
        pltpu.sync_copy(seli, pix_hbm)
        pltpu.sync_copy(scb, sc_hbm)


def _sc_topk(keys_flat):
    mesh = plsc.VectorSubcoreMesh(core_axis_name="c", subcore_axis_name="s")
    f = pl.kernel(
        _sc_body,
        out_type=[jax.ShapeDtypeStruct((128,), jnp.int32),
                  jax.ShapeDtypeStruct((128,), jnp.int32),
                  jax.ShapeDtypeStruct((128,), jnp.float32)],
        mesh=mesh,
        compiler_params=pltpu.CompilerParams(needs_layout_passes=False),
        scratch_types=[
            pltpu.VMEM((_CHUNK,), jnp.int32),
            pltpu.VMEM((_CHUNK,), jnp.int32),
            pltpu.VMEM((_NG,), jnp.int32),
            pltpu.VMEM((32,), jnp.int32),
            pltpu.VMEM((128,), jnp.int32),
            pltpu.VMEM((128,), jnp.int32),
            pltpu.VMEM_SHARED((2048,), jnp.int32),
            pltpu.VMEM_SHARED((2048,), jnp.int32),
            pltpu.VMEM((2048,), jnp.int32),
            pltpu.VMEM((2048,), jnp.int32),
            pltpu.VMEM((128,), jnp.int32),
            pltpu.VMEM((128,), jnp.int32),
            pltpu.VMEM((128,), jnp.int32),
            pltpu.VMEM((128,), jnp.float32),
        ],
    )
    return f(keys_flat)


def _boxes_kernel(pix_ref, pwh_ref, pxy_ref, boxes_ref):
    lane1 = lax.broadcasted_iota(jnp.int32, (1, 128), 1)
    pix = pix_ref[...]
    zf = jnp.zeros((1, 128), jnp.float32)

    def body(i, carry):
        bl, bt, br, bb = carry
        flat = jnp.sum(jnp.where(lane1 == i, pix, 0))
        b = flat // (_H * _W)
        rem = flat - b * (_H * _W)
        y = rem // _W
        xcol = rem - y * _W
        sel = lane1 == xcol
        rw = pwh_ref[pl.ds(b * 2 * _H + y, 1), :]
        rh = pwh_ref[pl.ds((b * 2 + 1) * _H + y, 1), :]
        rx = pxy_ref[pl.ds(b * 2 * _H + y, 1), :]
        ry = pxy_ref[pl.ds((b * 2 + 1) * _H + y, 1), :]
        w = jnp.sum(jnp.where(sel, rw, zf))
        h = jnp.sum(jnp.where(sel, rh, zf))
        ox = jnp.sum(jnp.where(sel, rx, zf))
        oy = jnp.sum(jnp.where(sel, ry, zf))
        xc = (ox + xcol.astype(jnp.float32)) / jnp.float32(_W)
        yc = (oy + y.astype(jnp.float32)) / jnp.float32(_H)
        oh = lane1 == i
        bl = jnp.where(oh, xc - w * 0.5, bl)
        bt = jnp.where(oh, yc - h * 0.5, bt)
        br = jnp.where(oh, xc + w * 0.5, br)
        bb = jnp.where(oh, yc + h * 0.5, bb)
        return bl, bt, br, bb

    bl, bt, br, bb = lax.fori_loop(0, _TOPK, body, (zf, zf, zf, zf))
    boxes_ref[...] = jnp.concatenate([bl, bt, br, bb], axis=0)


def kernel(pheatmap, pwh, pxy_offset, pkeypoint_offset):
    del pkeypoint_offset
    keys = pl.pallas_call(
        _nms_kernel,
        grid=(_B, _NCB),
        in_specs=[pl.BlockSpec((1, _CBLK, _H, _W), lambda b, c: (b, c, 0, 0))],
        out_specs=pl.BlockSpec((1, _H, _W), lambda b, c: (b, 0, 0)),
        out_shape=jax.ShapeDtypeStruct((_B, _H, _W), jnp.int32),
        scratch_shapes=[pltpu.VMEM((_H, _W), jnp.float32)],
    )(pheatmap)

    ids_p, pix_p, sc_p = _sc_topk(keys.reshape(-1))

    boxes_p = pl.pallas_call(
        _boxes_kernel,
        out_shape=jax.ShapeDtypeStruct((4, 128), jnp.float32),
    )(pix_p.reshape(1, 128), pwh.reshape(2 * _B * _H, _W),
      pxy_offset.reshape(2 * _B * _H, _W))

    ids = ids_p[:_TOPK]
    scores = sc_p[:_TOPK]
    boxes = boxes_p[:, :_TOPK].T
    return ids, boxes, scores, scores


# speedup vs baseline: 5.9783x; 1.1682x over previous
import functools
import jax
import jax.numpy as jnp
from jax import lax
from jax.experimental import pallas as pl
from jax.experimental.pallas import tpu as pltpu
from jax.experimental.pallas import tpu_sc as plsc

_THRESHOLD = 0.18
_TOPK = 100
_B, _C, _H, _W = 8, 80, 128, 128
_MIN = -2147483648
_CBLK = 16
_NCB = _C // _CBLK
_NW = 16
_CHUNK = (_B * _H * _W) // _NW
_NG = _CHUNK // 16
_NJ = _NG // 16


def _to_key(x):
    b = lax.bitcast_convert_type(x, jnp.int32)
    return jnp.where(b >= 0, b, b ^ jnp.int32(0x7FFFFFFF))


def _nms_kernel(x_ref, out_ref, acc_ref):
    c = pl.program_id(1)
    x = x_ref[0]
    lane = lax.broadcasted_iota(jnp.int32, (_CBLK, _H, _W), 2)
    sub = lax.broadcasted_iota(jnp.int32, (_CBLK, _H, _W), 1)
    ninf = jnp.float32(-jnp.inf)
    sr = jnp.where(lane == 0, ninf, pltpu.roll(x, 1, 2))
    sl = jnp.where(lane == _W - 1, ninf, pltpu.roll(x, _W - 1, 2))
    m1 = jnp.maximum(jnp.maximum(x, sr), sl)
    su = jnp.where(sub == 0, ninf, pltpu.roll(m1, 1, 1))
    sd = jnp.where(sub == _H - 1, ninf, pltpu.roll(m1, _H - 1, 1))
    m2 = jnp.maximum(jnp.maximum(m1, su), sd)
    cand = jnp.max(jnp.where(m2 == x, x, jnp.float32(0.0)), axis=0)

    @pl.when(c == 0)
    def _():
        acc_ref[...] = cand

    @pl.when(c > 0)
    def _():
        acc_ref[...] = jnp.maximum(acc_ref[...], cand)

    @pl.when(c == _NCB - 1)
    def _():
        acc = jnp.maximum(acc_ref[...], cand)
        masked = jnp.where(acc > _THRESHOLD, acc, ninf)
        out_ref[0] = _to_key(masked)


def _iota16():
    return lax.iota(jnp.int32, 16)


def _rmw_lane(ref, pos, val):
    base = (pos // 16) * 16
    lane = pos - base
    v = ref[pl.ds(base, 16)]
    ref[pl.ds(base, 16)] = jnp.where(_iota16() == lane, val, v)


def _sc_body(keys_hbm, ids_hbm, pix_hbm, sc_hbm,
             kv, cmx, gmax, g2, lk, li, shk, shi, mk, mi,
             selk, seli, idsb, scb):
    cid = lax.axis_index("c")
    wid = lax.axis_index("s")
    active = cid == 0
    w0 = jnp.logical_and(active, wid == 0)
    it = _iota16()

    @pl.when(active)
    def _():
        pltpu.sync_copy(keys_hbm.at[pl.ds(wid * _CHUNK, _CHUNK)], kv)
        def p1(g, _):
            cmx[pl.ds(g * 16, 16)] = plsc.cummax(kv[pl.ds(g * 16, 16)])
            return 0
        lax.fori_loop(0, _NG, p1, 0)

        def p1b(j, _):
            gmax[pl.ds(j * 16, 16)] = plsc.load_gather(
                cmx, [j * 256 + it * 16 + 15])
            return 0
        lax.fori_loop(0, _NJ, p1b, 0)

        def p1c(j, _):
            cmx[pl.ds(j * 16, 16)] = plsc.cummax(gmax[pl.ds(j * 16, 16)])
            return 0
        lax.fori_loop(0, _NJ, p1c, 0)
        g2[pl.ds(0, 16)] = plsc.load_gather(cmx, [it * 16 + 15])
        g2[pl.ds(16, 16)] = plsc.load_gather(cmx, [256 + it * 16 + 15])

        def p2(r, _):
            va = g2[pl.ds(0, 16)]
            vb = g2[pl.ds(16, 16)]
            take = vb > va
            cv = jnp.where(take, vb, va)
            cj = jnp.where(take, it + 16, it)
            m = jnp.max(cv)
            jstar = jnp.min(jnp.where(cv == m, cj, jnp.int32(31)))
            gv = gmax[pl.ds(jstar * 16, 16)]
            gin = jnp.min(jnp.where(gv == m, it, jnp.int32(15)))
            gstar = jstar * 16 + gin
            kvv = kv[pl.ds(gstar * 16, 16)]
            lstar = jnp.min(jnp.where(kvv == m, it, jnp.int32(15)))
            lidx = gstar * 16 + lstar

            kvv2 = jnp.where(it == lstar, jnp.int32(_MIN), kvv)
            kv[pl.ds(gstar * 16, 16)] = kvv2
            gv2 = jnp.where(it == gin, jnp.max(kvv2), gv)
            gmax[pl.ds(jstar * 16, 16)] = gv2
            ng2 = jnp.max(gv2)
            base2 = (jstar // 16) * 16
            v2 = g2[pl.ds(base2, 16)]
            g2[pl.ds(base2, 16)] = jnp.where(it == jstar - base2, ng2, v2)

            _rmw_lane(lk, r, m)
            _rmw_lane(li, r, wid * _CHUNK + lidx)
            return 0
        lax.fori_loop(0, _TOPK, p2, 0)

        for t in range(_TOPK // 16, 8):
            padmask = (t * 16 + it) >= _TOPK
            lk[pl.ds(t * 16, 16)] = jnp.where(padmask, jnp.int32(_MIN),
                                              lk[pl.ds(t * 16, 16)])
            li[pl.ds(t * 16, 16)] = jnp.where(padmask, jnp.int32(0),
                                              li[pl.ds(t * 16, 16)])
        pltpu.sync_copy(lk, shk.at[pl.ds(wid * 128, 128)])
        pltpu.sync_copy(li, shi.at[pl.ds(wid * 128, 128)])

    plsc.subcore_barrier()

    @pl.when(w0)
    def _():
        pltpu.sync_copy(shk, mk)
        pltpu.sync_copy(shi, mi)
        heads0 = plsc.load_gather(mk, [it * 128])
        ptrs0 = jnp.zeros((16,), jnp.int32)

        def p4(i, carry):
            heads, ptrs = carry
            m = jnp.max(heads)
            wstar = jnp.min(jnp.where(heads == m, it, jnp.int32(15)))
            p = jnp.minimum(
                jnp.min(jnp.where(it == wstar, ptrs, jnp.int32(126))),
                jnp.int32(126))
            gidx = mi[pl.ds(wstar * 128 + p, 16)][0]
            nxt = mk[pl.ds(wstar * 128 + p + 1, 16)][0]
            heads = jnp.where(it == wstar, nxt, heads)
            ptrs = jnp.where(it == wstar, p + 1, ptrs)
            _rmw_lane(selk, i, m)
            _rmw_lane(seli, i, gidx)
            return heads, ptrs
        lax.fori_loop(0, _TOPK, p4, (heads0, ptrs0))

        for t in range(8):
            sl = pl.ds(t * 16, 16)
            iv = seli[sl]
            idsb[sl] = lax.shift_right_arithmetic(iv, 14)
            k = selk[sl]
            bits = jnp.where(k >= 0, k, k ^ jnp.int32(0x7FFFFFFF))
            scb[sl] = plsc.bitcast(bits, jnp.float32)

        pltpu.sync_copy(idsb, ids_hbm)
        pltpu.sync_copy(seli, pix_hbm)
        pltpu.sync_copy(scb, sc_hbm)


def _sc_topk(keys_flat):
    mesh = plsc.VectorSubcoreMesh(core_axis_name="c", subcore_axis_name="s")
    f = pl.kernel(
        _sc_body,
        out_type=[jax.ShapeDtypeStruct((128,), jnp.int32),
                  jax.ShapeDtypeStruct((128,), jnp.int32),
                  jax.ShapeDtypeStruct((128,), jnp.float32)],
        mesh=mesh,
        compiler_params=pltpu.CompilerParams(needs_layout_passes=False),
        scratch_types=[
            pltpu.VMEM((_CHUNK,), jnp.int32),
            pltpu.VMEM((_CHUNK,), jnp.int32),
            pltpu.VMEM((_NG,), jnp.int32),
            pltpu.VMEM((32,), jnp.int32),
            pltpu.VMEM((128,), jnp.int32),
            pltpu.VMEM((128,), jnp.int32),
            pltpu.VMEM_SHARED((2048,), jnp.int32),
            pltpu.VMEM_SHARED((2048,), jnp.int32),
            pltpu.VMEM((2048,), jnp.int32),
            pltpu.VMEM((2048,), jnp.int32),
            pltpu.VMEM((128,), jnp.int32),
            pltpu.VMEM((128,), jnp.int32),
            pltpu.VMEM((128,), jnp.int32),
            pltpu.VMEM((128,), jnp.float32),
        ],
    )
    return f(keys_flat)


def _boxes_kernel(pix_ref, pwh_ref, pxy_ref, boxes_ref):
    lane1 = lax.broadcasted_iota(jnp.int32, (1, 128), 1)
    pix = pix_ref[...]
    zf = jnp.zeros((1, 128), jnp.float32)

    def body(i, carry):
        bl, bt, br, bb = carry
        flat = jnp.sum(jnp.where(lane1 == i, pix, 0))
        b = flat // (_H * _W)
        rem = flat - b * (_H * _W)
        y = rem // _W
        xcol = rem - y * _W
        sel = lane1 == xcol
        rw = pwh_ref[pl.ds(b * 2 * _H + y, 1), :]
        rh = pwh_ref[pl.ds((b * 2 + 1) * _H + y, 1), :]
        rx = pxy_ref[pl.ds(b * 2 * _H + y, 1), :]
        ry = pxy_ref[pl.ds((b * 2 + 1) * _H + y, 1), :]
        w = jnp.sum(jnp.where(sel, rw, zf))
        h = jnp.sum(jnp.where(sel, rh, zf))
        ox = jnp.sum(jnp.where(sel, rx, zf))
        oy = jnp.sum(jnp.where(sel, ry, zf))
        xc = (ox + xcol.astype(jnp.float32)) / jnp.float32(_W)
        yc = (oy + y.astype(jnp.float32)) / jnp.float32(_H)
        oh = lane1 == i
        bl = jnp.where(oh, xc - w * 0.5, bl)
        bt = jnp.where(oh, yc - h * 0.5, bt)
        br = jnp.where(oh, xc + w * 0.5, br)
        bb = jnp.where(oh, yc + h * 0.5, bb)
        return bl, bt, br, bb

    bl, bt, br, bb = lax.fori_loop(0, _TOPK, body, (zf, zf, zf, zf))
    boxes_ref[...] = jnp.concatenate([bl, bt, br, bb], axis=0)


def kernel(pheatmap, pwh, pxy_offset, pkeypoint_offset):
    del pkeypoint_offset
    keys = pl.pallas_call(
        _nms_kernel,
        grid=(_B, _NCB),
        in_specs=[pl.BlockSpec((1, _CBLK, _H, _W), lambda b, c: (b, c, 0, 0))],
        out_specs=pl.BlockSpec((1, _H, _W), lambda b, c: (b, 0, 0)),
        out_shape=jax.ShapeDtypeStruct((_B, _H, _W), jnp.int32),
        scratch_shapes=[pltpu.VMEM((_H, _W), jnp.float32)],
    )(pheatmap)

    ids_p, pix_p, sc_p = _sc_topk(keys.reshape(-1))

    boxes_p = pl.pallas_call(
        _boxes_kernel,
        out_shape=jax.ShapeDtypeStruct((4, 128), jnp.float32),
    )(pix_p.reshape(1, 128), pwh.reshape(2 * _B * _H, _W),
      pxy_offset.reshape(2 * _B * _H, _W))

    ids = ids_p[:_TOPK]
    scores = sc_p[:_TOPK]
    boxes = boxes_p[:, :_TOPK].T
    return ids, boxes, scores, scores


# speedup vs baseline: 6.4824x; 1.0843x over previous
import functools
import jax
import jax.numpy as jnp
from jax import lax
from jax.experimental import pallas as pl
from jax.experimental.pallas import tpu as pltpu
from jax.experimental.pallas import tpu_sc as plsc

_THRESHOLD = 0.18
_TOPK = 100
_B, _C, _H, _W = 8, 80, 128, 128
_MIN = -2147483648
_CBLK = 40
_NCB = _C // _CBLK
_NW = 16
_CHUNK = (_B * _H * _W) // _NW
_NG = _CHUNK // 16
_NJ = _NG // 16


def _to_key(x):
    b = lax.bitcast_convert_type(x, jnp.int32)
    return jnp.where(b >= 0, b, b ^ jnp.int32(0x7FFFFFFF))


def _nms_kernel(x_ref, out_ref, acc_ref):
    c = pl.program_id(1)
    x = x_ref[0]
    lane = lax.broadcasted_iota(jnp.int32, (_CBLK, _H, _W), 2)
    sub = lax.broadcasted_iota(jnp.int32, (_CBLK, _H, _W), 1)
    ninf = jnp.float32(-jnp.inf)
    sr = jnp.where(lane == 0, ninf, pltpu.roll(x, 1, 2))
    sl = jnp.where(lane == _W - 1, ninf, pltpu.roll(x, _W - 1, 2))
    m1 = jnp.maximum(jnp.maximum(x, sr), sl)
    su = jnp.where(sub == 0, ninf, pltpu.roll(m1, 1, 1))
    sd = jnp.where(sub == _H - 1, ninf, pltpu.roll(m1, _H - 1, 1))
    m2 = jnp.maximum(jnp.maximum(m1, su), sd)
    cand = jnp.max(jnp.where(m2 == x, x, jnp.float32(0.0)), axis=0)

    @pl.when(c == 0)
    def _():
        acc_ref[...] = cand

    @pl.when(c > 0)
    def _():
        acc_ref[...] = jnp.maximum(acc_ref[...], cand)

    @pl.when(c == _NCB - 1)
    def _():
        acc = jnp.maximum(acc_ref[...], cand)
        masked = jnp.where(acc > _THRESHOLD, acc, ninf)
        out_ref[0] = _to_key(masked)


def _iota16():
    return lax.iota(jnp.int32, 16)


def _rmw_lane(ref, pos, val):
    base = (pos // 16) * 16
    lane = pos - base
    v = ref[pl.ds(base, 16)]
    ref[pl.ds(base, 16)] = jnp.where(_iota16() == lane, val, v)


def _sc_body(keys_hbm, ids_hbm, pix_hbm, sc_hbm,
             kv, cmx, gmax, g2, lk, li, shk, shi, mk, mi,
             selk, seli, idsb, scb):
    cid = lax.axis_index("c")
    wid = lax.axis_index("s")
    active = cid == 0
    w0 = jnp.logical_and(active, wid == 0)
    it = _iota16()

    @pl.when(active)
    def _():
        pltpu.sync_copy(keys_hbm.at[pl.ds(wid * _CHUNK, _CHUNK)], kv)
        def p1(g, _):
            cmx[pl.ds(g * 16, 16)] = plsc.cummax(kv[pl.ds(g * 16, 16)])
            return 0
        lax.fori_loop(0, _NG, p1, 0)

        def p1b(j, _):
            gmax[pl.ds(j * 16, 16)] = plsc.load_gather(
                cmx, [j * 256 + it * 16 + 15])
            return 0
        lax.fori_loop(0, _NJ, p1b, 0)

        def p1c(j, _):
            cmx[pl.ds(j * 16, 16)] = plsc.cummax(gmax[pl.ds(j * 16, 16)])
            return 0
        lax.fori_loop(0, _NJ, p1c, 0)
        g2[pl.ds(0, 16)] = plsc.load_gather(cmx, [it * 16 + 15])
        g2[pl.ds(16, 16)] = plsc.load_gather(cmx, [256 + it * 16 + 15])

        def p2(r, _):
            va = g2[pl.ds(0, 16)]
            vb = g2[pl.ds(16, 16)]
            take = vb > va
            cv = jnp.where(take, vb, va)
            cj = jnp.where(take, it + 16, it)
            m = jnp.max(cv)
            jstar = jnp.min(jnp.where(cv == m, cj, jnp.int32(31)))
            gv = gmax[pl.ds(jstar * 16, 16)]
            gin = jnp.min(jnp.where(gv == m, it, jnp.int32(15)))
            gstar = jstar * 16 + gin
            kvv = kv[pl.ds(gstar * 16, 16)]
            lstar = jnp.min(jnp.where(kvv == m, it, jnp.int32(15)))
            lidx = gstar * 16 + lstar

            kvv2 = jnp.where(it == lstar, jnp.int32(_MIN), kvv)
            kv[pl.ds(gstar * 16, 16)] = kvv2
            gv2 = jnp.where(it == gin, jnp.max(kvv2), gv)
            gmax[pl.ds(jstar * 16, 16)] = gv2
            ng2 = jnp.max(gv2)
            base2 = (jstar // 16) * 16
            v2 = g2[pl.ds(base2, 16)]
            g2[pl.ds(base2, 16)] = jnp.where(it == jstar - base2, ng2, v2)

            _rmw_lane(lk, r, m)
            _rmw_lane(li, r, wid * _CHUNK + lidx)
            return 0
        lax.fori_loop(0, _TOPK, p2, 0)

        for t in range(_TOPK // 16, 8):
            padmask = (t * 16 + it) >= _TOPK
            lk[pl.ds(t * 16, 16)] = jnp.where(padmask, jnp.int32(_MIN),
                                              lk[pl.ds(t * 16, 16)])
            li[pl.ds(t * 16, 16)] = jnp.where(padmask, jnp.int32(0),
                                              li[pl.ds(t * 16, 16)])
        pltpu.sync_copy(lk, shk.at[pl.ds(wid * 128, 128)])
        pltpu.sync_copy(li, shi.at[pl.ds(wid * 128, 128)])

    plsc.subcore_barrier()

    @pl.when(w0)
    def _():
        pltpu.sync_copy(shk, mk)
        pltpu.sync_copy(shi, mi)
        heads0 = plsc.load_gather(mk, [it * 128])
        ptrs0 = jnp.zeros((16,), jnp.int32)

        def p4(i, carry):
            heads, ptrs = carry
            m = jnp.max(heads)
            wstar = jnp.min(jnp.where(heads == m, it, jnp.int32(15)))
            p = jnp.minimum(
                jnp.min(jnp.where(it == wstar, ptrs, jnp.int32(126))),
                jnp.int32(126))
            gidx = mi[pl.ds(wstar * 128 + p, 16)][0]
            nxt = mk[pl.ds(wstar * 128 + p + 1, 16)][0]
            heads = jnp.where(it == wstar, nxt, heads)
            ptrs = jnp.where(it == wstar, p + 1, ptrs)
            _rmw_lane(selk, i, m)
            _rmw_lane(seli, i, gidx)
            return heads, ptrs
        lax.fori_loop(0, _TOPK, p4, (heads0, ptrs0))

        for t in range(8):
            sl = pl.ds(t * 16, 16)
            iv = seli[sl]
            idsb[sl] = lax.shift_right_arithmetic(iv, 14)
            k = selk[sl]
            bits = jnp.where(k >= 0, k, k ^ jnp.int32(0x7FFFFFFF))
            scb[sl] = plsc.bitcast(bits, jnp.float32)

        pltpu.sync_copy(idsb, ids_hbm)
        pltpu.sync_copy(seli, pix_hbm)
        pltpu.sync_copy(scb, sc_hbm)


def _sc_topk(keys_flat):
    mesh = plsc.VectorSubcoreMesh(core_axis_name="c", subcore_axis_name="s")
    f = pl.kernel(
        _sc_body,
        out_type=[jax.ShapeDtypeStruct((128,), jnp.int32),
                  jax.ShapeDtypeStruct((128,), jnp.int32),
                  jax.ShapeDtypeStruct((128,), jnp.float32)],
        mesh=mesh,
        compiler_params=pltpu.CompilerParams(needs_layout_passes=False),
        scratch_types=[
            pltpu.VMEM((_CHUNK,), jnp.int32),
            pltpu.VMEM((_CHUNK,), jnp.int32),
            pltpu.VMEM((_NG,), jnp.int32),
            pltpu.VMEM((32,), jnp.int32),
            pltpu.VMEM((128,), jnp.int32),
            pltpu.VMEM((128,), jnp.int32),
            pltpu.VMEM_SHARED((2048,), jnp.int32),
            pltpu.VMEM_SHARED((2048,), jnp.int32),
            pltpu.VMEM((2048,), jnp.int32),
            pltpu.VMEM((2048,), jnp.int32),
            pltpu.VMEM((128,), jnp.int32),
            pltpu.VMEM((128,), jnp.int32),
            pltpu.VMEM((128,), jnp.int32),
            pltpu.VMEM((128,), jnp.float32),
        ],
    )
    return f(keys_flat)


def _boxes_kernel(pix_ref, pwh_ref, pxy_ref, boxes_ref):
    lane1 = lax.broadcasted_iota(jnp.int32, (1, 128), 1)
    pix = pix_ref[...]
    zf = jnp.zeros((1, 128), jnp.float32)

    def body(i, carry):
        bl, bt, br, bb = carry
        flat = jnp.sum(jnp.where(lane1 == i, pix, 0))
        b = flat // (_H * _W)
        rem = flat - b * (_H * _W)
        y = rem // _W
        xcol = rem - y * _W
        sel = lane1 == xcol
        rw = pwh_ref[pl.ds(b * 2 * _H + y, 1), :]
        rh = pwh_ref[pl.ds((b * 2 + 1) * _H + y, 1), :]
        rx = pxy_ref[pl.ds(b * 2 * _H + y, 1), :]
        ry = pxy_ref[pl.ds((b * 2 + 1) * _H + y, 1), :]
        w = jnp.sum(jnp.where(sel, rw, zf))
        h = jnp.sum(jnp.where(sel, rh, zf))
        ox = jnp.sum(jnp.where(sel, rx, zf))
        oy = jnp.sum(jnp.where(sel, ry, zf))
        xc = (ox + xcol.astype(jnp.float32)) / jnp.float32(_W)
        yc = (oy + y.astype(jnp.float32)) / jnp.float32(_H)
        oh = lane1 == i
        bl = jnp.where(oh, xc - w * 0.5, bl)
        bt = jnp.where(oh, yc - h * 0.5, bt)
        br = jnp.where(oh, xc + w * 0.5, br)
        bb = jnp.where(oh, yc + h * 0.5, bb)
        return bl, bt, br, bb

    bl, bt, br, bb = lax.fori_loop(0, _TOPK, body, (zf, zf, zf, zf))
    boxes_ref[...] = jnp.concatenate([bl, bt, br, bb], axis=0)


def kernel(pheatmap, pwh, pxy_offset, pkeypoint_offset):
    del pkeypoint_offset
    keys = pl.pallas_call(
        _nms_kernel,
        grid=(_B, _NCB),
        in_specs=[pl.BlockSpec((1, _CBLK, _H, _W), lambda b, c: (b, c, 0, 0))],
        out_specs=pl.BlockSpec((1, _H, _W), lambda b, c: (b, 0, 0)),
        out_shape=jax.ShapeDtypeStruct((_B, _H, _W), jnp.int32),
        scratch_shapes=[pltpu.VMEM((_H, _W), jnp.float32)],
    )(pheatmap)

    ids_p, pix_p, sc_p = _sc_topk(keys.reshape(-1))

    boxes_p = pl.pallas_call(
        _boxes_kernel,
        out_shape=jax.ShapeDtypeStruct((4, 128), jnp.float32),
    )(pix_p.reshape(1, 128), pwh.reshape(2 * _B * _H, _W),
      pxy_offset.reshape(2 * _B * _H, _W))

    ids = ids_p[:_TOPK]
    scores = sc_p[:_TOPK]
    boxes = boxes_p[:, :_TOPK].T
    return ids, boxes, scores, scores


# speedup vs baseline: 6.5134x; 1.0048x over previous
import functools
import jax
import jax.numpy as jnp
from jax import lax
from jax.experimental import pallas as pl
from jax.experimental.pallas import tpu as pltpu
from jax.experimental.pallas import tpu_sc as plsc

_THRESHOLD = 0.18
_TOPK = 100
_B, _C, _H, _W = 8, 80, 128, 128
_MIN = -2147483648
_CBLK = 80
_NCB = _C // _CBLK
_NW = 16
_CHUNK = (_B * _H * _W) // _NW
_NG = _CHUNK // 16
_NJ = _NG // 16


def _to_key(x):
    b = lax.bitcast_convert_type(x, jnp.int32)
    return jnp.where(b >= 0, b, b ^ jnp.int32(0x7FFFFFFF))


def _nms_kernel(x_ref, out_ref, acc_ref):
    c = pl.program_id(1)
    x = x_ref[0]
    lane = lax.broadcasted_iota(jnp.int32, (_CBLK, _H, _W), 2)
    sub = lax.broadcasted_iota(jnp.int32, (_CBLK, _H, _W), 1)
    ninf = jnp.float32(-jnp.inf)
    sr = jnp.where(lane == 0, ninf, pltpu.roll(x, 1, 2))
    sl = jnp.where(lane == _W - 1, ninf, pltpu.roll(x, _W - 1, 2))
    m1 = jnp.maximum(jnp.maximum(x, sr), sl)
    su = jnp.where(sub == 0, ninf, pltpu.roll(m1, 1, 1))
    sd = jnp.where(sub == _H - 1, ninf, pltpu.roll(m1, _H - 1, 1))
    m2 = jnp.maximum(jnp.maximum(m1, su), sd)
    cand = jnp.max(jnp.where(m2 == x, x, jnp.float32(0.0)), axis=0)

    @pl.when(c == 0)
    def _():
        acc_ref[...] = cand

    @pl.when(c > 0)
    def _():
        acc_ref[...] = jnp.maximum(acc_ref[...], cand)

    @pl.when(c == _NCB - 1)
    def _():
        acc = jnp.maximum(acc_ref[...], cand)
        masked = jnp.where(acc > _THRESHOLD, acc, ninf)
        out_ref[0] = _to_key(masked)


def _iota16():
    return lax.iota(jnp.int32, 16)


def _rmw_lane(ref, pos, val):
    base = (pos // 16) * 16
    lane = pos - base
    v = ref[pl.ds(base, 16)]
    ref[pl.ds(base, 16)] = jnp.where(_iota16() == lane, val, v)


def _sc_body(keys_hbm, ids_hbm, pix_hbm, sc_hbm,
             kv, cmx, gmax, g2, lk, li, shk, shi, mk, mi,
             selk, seli, idsb, scb):
    cid = lax.axis_index("c")
    wid = lax.axis_index("s")
    active = cid == 0
    w0 = jnp.logical_and(active, wid == 0)
    it = _iota16()

    @pl.when(active)
    def _():
        pltpu.sync_copy(keys_hbm.at[pl.ds(wid * _CHUNK, _CHUNK)], kv)
        def p1(g, _):
            cmx[pl.ds(g * 16, 16)] = plsc.cummax(kv[pl.ds(g * 16, 16)])
            return 0
        lax.fori_loop(0, _NG, p1, 0)

        def p1b(j, _):
            gmax[pl.ds(j * 16, 16)] = plsc.load_gather(
                cmx, [j * 256 + it * 16 + 15])
            return 0
        lax.fori_loop(0, _NJ, p1b, 0)

        def p1c(j, _):
            cmx[pl.ds(j * 16, 16)] = plsc.cummax(gmax[pl.ds(j * 16, 16)])
            return 0
        lax.fori_loop(0, _NJ, p1c, 0)
        g2[pl.ds(0, 16)] = plsc.load_gather(cmx, [it * 16 + 15])
        g2[pl.ds(16, 16)] = plsc.load_gather(cmx, [256 + it * 16 + 15])

        def p2(r, _):
            va = g2[pl.ds(0, 16)]
            vb = g2[pl.ds(16, 16)]
            take = vb > va
            cv = jnp.where(take, vb, va)
            cj = jnp.where(take, it + 16, it)
            m = jnp.max(cv)
            jstar = jnp.min(jnp.where(cv == m, cj, jnp.int32(31)))
            gv = gmax[pl.ds(jstar * 16, 16)]
            gin = jnp.min(jnp.where(gv == m, it, jnp.int32(15)))
            gstar = jstar * 16 + gin
            kvv = kv[pl.ds(gstar * 16, 16)]
            lstar = jnp.min(jnp.where(kvv == m, it, jnp.int32(15)))
            lidx = gstar * 16 + lstar

            kvv2 = jnp.where(it == lstar, jnp.int32(_MIN), kvv)
            kv[pl.ds(gstar * 16, 16)] = kvv2
            gv2 = jnp.where(it == gin, jnp.max(kvv2), gv)
            gmax[pl.ds(jstar * 16, 16)] = gv2
            ng2 = jnp.max(gv2)
            base2 = (jstar // 16) * 16
            v2 = g2[pl.ds(base2, 16)]
            g2[pl.ds(base2, 16)] = jnp.where(it == jstar - base2, ng2, v2)

            _rmw_lane(lk, r, m)
            _rmw_lane(li, r, wid * _CHUNK + lidx)
            return 0
        lax.fori_loop(0, _TOPK, p2, 0)

        for t in range(_TOPK // 16, 8):
            padmask = (t * 16 + it) >= _TOPK
            lk[pl.ds(t * 16, 16)] = jnp.where(padmask, jnp.int32(_MIN),
                                              lk[pl.ds(t * 16, 16)])
            li[pl.ds(t * 16, 16)] = jnp.where(padmask, jnp.int32(0),
                                              li[pl.ds(t * 16, 16)])
        pltpu.sync_copy(lk, shk.at[pl.ds(wid * 128, 128)])
        pltpu.sync_copy(li, shi.at[pl.ds(wid * 128, 128)])

    plsc.subcore_barrier()

    @pl.when(w0)
    def _():
        pltpu.sync_copy(shk, mk)
        pltpu.sync_copy(shi, mi)
        heads0 = plsc.load_gather(mk, [it * 128])
        ptrs0 = jnp.zeros((16,), jnp.int32)

        def p4(i, carry):
            heads, ptrs = carry
            m = jnp.max(heads)
            wstar = jnp.min(jnp.where(heads == m, it, jnp.int32(15)))
            p = jnp.minimum(
                jnp.min(jnp.where(it == wstar, ptrs, jnp.int32(126))),
                jnp.int32(126))
            gidx = mi[pl.ds(wstar * 128 + p, 16)][0]
            nxt = mk[pl.ds(wstar * 128 + p + 1, 16)][0]
            heads = jnp.where(it == wstar, nxt, heads)
            ptrs = jnp.where(it == wstar, p + 1, ptrs)
            _rmw_lane(selk, i, m)
            _rmw_lane(seli, i, gidx)
            return heads, ptrs
        lax.fori_loop(0, _TOPK, p4, (heads0, ptrs0))

        for t in range(8):
            sl = pl.ds(t * 16, 16)
            iv = seli[sl]
            idsb[sl] = lax.shift_right_arithmetic(iv, 14)
            k = selk[sl]
            bits = jnp.where(k >= 0, k, k ^ jnp.int32(0x7FFFFFFF))
            scb[sl] = plsc.bitcast(bits, jnp.float32)

        pltpu.sync_copy(idsb, ids_hbm)
        pltpu.sync_copy(seli, pix_hbm)
        pltpu.sync_copy(scb, sc_hbm)


def _sc_topk(keys_flat):
    mesh = plsc.VectorSubcoreMesh(core_axis_name="c", subcore_axis_name="s")
    f = pl.kernel(
        _sc_body,
        out_type=[jax.ShapeDtypeStruct((128,), jnp.int32),
                  jax.ShapeDtypeStruct((128,), jnp.int32),
                  jax.ShapeDtypeStruct((128,), jnp.float32)],
        mesh=mesh,
        compiler_params=pltpu.CompilerParams(needs_layout_passes=False),
        scratch_types=[
            pltpu.VMEM((_CHUNK,), jnp.int32),
            pltpu.VMEM((_CHUNK,), jnp.int32),
            pltpu.VMEM((_NG,), jnp.int32),
            pltpu.VMEM((32,), jnp.int32),
            pltpu.VMEM((128,), jnp.int32),
            pltpu.VMEM((128,), jnp.int32),
            pltpu.VMEM_SHARED((2048,), jnp.int32),
            pltpu.VMEM_SHARED((2048,), jnp.int32),
            pltpu.VMEM((2048,), jnp.int32),
            pltpu.VMEM((2048,), jnp.int32),
            pltpu.VMEM((128,), jnp.int32),
            pltpu.VMEM((128,), jnp.int32),
            pltpu.VMEM((128,), jnp.int32),
            pltpu.VMEM((128,), jnp.float32),
        ],
    )
    return f(keys_flat)


def _boxes_kernel(pix_ref, pwh_ref, pxy_ref, boxes_ref):
    lane1 = lax.broadcasted_iota(jnp.int32, (1, 128), 1)
    pix = pix_ref[...]
    zf = jnp.zeros((1, 128), jnp.float32)

    def body(i, carry):
        bl, bt, br, bb = carry
        flat = jnp.sum(jnp.where(lane1 == i, pix, 0))
        b = flat // (_H * _W)
        rem = flat - b * (_H * _W)
        y = rem // _W
        xcol = rem - y * _W
        sel = lane1 == xcol
        rw = pwh_ref[pl.ds(b * 2 * _H + y, 1), :]
        rh = pwh_ref[pl.ds((b * 2 + 1) * _H + y, 1), :]
        rx = pxy_ref[pl.ds(b * 2 * _H + y, 1), :]
        ry = pxy_ref[pl.ds((b * 2 + 1) * _H + y, 1), :]
        w = jnp.sum(jnp.where(sel, rw, zf))
        h = jnp.sum(jnp.where(sel, rh, zf))
        ox = jnp.sum(jnp.where(sel, rx, zf))
        oy = jnp.sum(jnp.where(sel, ry, zf))
        xc = (ox + xcol.astype(jnp.float32)) / jnp.float32(_W)
        yc = (oy + y.astype(jnp.float32)) / jnp.float32(_H)
        oh = lane1 == i
        bl = jnp.where(oh, xc - w * 0.5, bl)
        bt = jnp.where(oh, yc - h * 0.5, bt)
        br = jnp.where(oh, xc + w * 0.5, br)
        bb = jnp.where(oh, yc + h * 0.5, bb)
        return bl, bt, br, bb

    bl, bt, br, bb = lax.fori_loop(0, _TOPK, body, (zf, zf, zf, zf))
    boxes_ref[...] = jnp.concatenate([bl, bt, br, bb], axis=0)


def kernel(pheatmap, pwh, pxy_offset, pkeypoint_offset):
    del pkeypoint_offset
    keys = pl.pallas_call(
        _nms_kernel,
        grid=(_B, _NCB),
        in_specs=[pl.BlockSpec((1, _CBLK, _H, _W), lambda b, c: (b, c, 0, 0))],
        out_specs=pl.BlockSpec((1, _H, _W), lambda b, c: (b, 0, 0)),
        out_shape=jax.ShapeDtypeStruct((_B, _H, _W), jnp.int32),
        scratch_shapes=[pltpu.VMEM((_H, _W), jnp.float32)],
    )(pheatmap)

    ids_p, pix_p, sc_p = _sc_topk(keys.reshape(-1))

    boxes_p = pl.pallas_call(
        _boxes_kernel,
        out_shape=jax.ShapeDtypeStruct((4, 128), jnp.float32),
    )(pix_p.reshape(1, 128), pwh.reshape(2 * _B * _H, _W),
      pxy_offset.reshape(2 * _B * _H, _W))

    ids = ids_p[:_TOPK]
    scores = sc_p[:_TOPK]
    boxes = boxes_p[:, :_TOPK].T
    return ids, boxes, scores, scores


# speedup vs baseline: 6.5150x; 1.0002x over previous
import jax
import jax.numpy as jnp
from jax import lax
from jax.experimental import pallas as pl
from jax.experimental.pallas import tpu as pltpu
from jax.experimental.pallas import tpu_sc as plsc

_THRESHOLD = 0.18
_TOPK = 100
_B, _C, _H, _W = 8, 80, 128, 128
_MIN = -2147483648
_CBLK = 80
_NCB = _C // _CBLK
_NW = 16
_CHUNK = (_B * _H * _W) // _NW
_NG = _CHUNK // 16
_NJ = _NG // 16


def _to_key(x):
    b = lax.bitcast_convert_type(x, jnp.int32)
    return jnp.where(b >= 0, b, b ^ jnp.int32(0x7FFFFFFF))


def _nms_kernel(x_ref, out_ref, acc_ref):
    c = pl.program_id(1)
    x = x_ref[0]
    lane = lax.broadcasted_iota(jnp.int32, (_CBLK, _H, _W), 2)
    sub = lax.broadcasted_iota(jnp.int32, (_CBLK, _H, _W), 1)
    ninf = jnp.float32(-jnp.inf)
    sr = jnp.where(lane == 0, ninf, pltpu.roll(x, 1, 2))
    sl = jnp.where(lane == _W - 1, ninf, pltpu.roll(x, _W - 1, 2))
    m1 = jnp.maximum(jnp.maximum(x, sr), sl)
    su = jnp.where(sub == 0, ninf, pltpu.roll(m1, 1, 1))
    sd = jnp.where(sub == _H - 1, ninf, pltpu.roll(m1, _H - 1, 1))
    m2 = jnp.maximum(jnp.maximum(m1, su), sd)
    cand = jnp.max(jnp.where(m2 == x, x, jnp.float32(0.0)), axis=0)

    @pl.when(c == 0)
    def _():
        acc_ref[...] = cand

    @pl.when(c > 0)
    def _():
        acc_ref[...] = jnp.maximum(acc_ref[...], cand)

    @pl.when(c == _NCB - 1)
    def _():
        acc = jnp.maximum(acc_ref[...], cand)
        masked = jnp.where(acc > _THRESHOLD, acc, ninf)
        out_ref[0] = _to_key(masked)


def _iota16():
    return lax.iota(jnp.int32, 16)


def _rmw_lane(ref, pos, val):
    base = (pos // 16) * 16
    lane = pos - base
    v = ref[pl.ds(base, 16)]
    ref[pl.ds(base, 16)] = jnp.where(_iota16() == lane, val, v)


def _sc_body(keys_hbm, ids_hbm, pix_hbm, sc_hbm,
             kv, cmx, gmax, g2, lk, li, shk, shi, mk, mi,
             selk, seli, idsb, scb):
    cid = lax.axis_index("c")
    wid = lax.axis_index("s")
    active = cid == 0
    w0 = jnp.logical_and(active, wid == 0)
    it = _iota16()

    @pl.when(active)
    def _():
        pltpu.sync_copy(keys_hbm.at[pl.ds(wid * _CHUNK, _CHUNK)], kv)
        def p1(g, _):
            cmx[pl.ds(g * 16, 16)] = plsc.cummax(kv[pl.ds(g * 16, 16)])
            return 0
        lax.fori_loop(0, _NG, p1, 0)

        def p1b(j, _):
            gmax[pl.ds(j * 16, 16)] = plsc.load_gather(
                cmx, [j * 256 + it * 16 + 15])
            return 0
        lax.fori_loop(0, _NJ, p1b, 0)

        def p1c(j, _):
            cmx[pl.ds(j * 16, 16)] = plsc.cummax(gmax[pl.ds(j * 16, 16)])
            return 0
        lax.fori_loop(0, _NJ, p1c, 0)
        g2[pl.ds(0, 16)] = plsc.load_gather(cmx, [it * 16 + 15])
        g2[pl.ds(16, 16)] = plsc.load_gather(cmx, [256 + it * 16 + 15])

        def p2(r, _):
            va = g2[pl.ds(0, 16)]
            vb = g2[pl.ds(16, 16)]
            take = vb > va
            cv = jnp.where(take, vb, va)
            cj = jnp.where(take, it + 16, it)
            m = jnp.max(cv)
            jstar = jnp.min(jnp.where(cv == m, cj, jnp.int32(31)))
            gv = gmax[pl.ds(jstar * 16, 16)]
            gin = jnp.min(jnp.where(gv == m, it, jnp.int32(15)))
            gstar = jstar * 16 + gin
            kvv = kv[pl.ds(gstar * 16, 16)]
            lstar = jnp.min(jnp.where(kvv == m, it, jnp.int32(15)))
            lidx = gstar * 16 + lstar

            kvv2 = jnp.where(it == lstar, jnp.int32(_MIN), kvv)
            kv[pl.ds(gstar * 16, 16)] = kvv2
            gv2 = jnp.where(it == gin, jnp.max(kvv2), gv)
            gmax[pl.ds(jstar * 16, 16)] = gv2
            ng2 = jnp.max(gv2)
            base2 = (jstar // 16) * 16
            v2 = g2[pl.ds(base2, 16)]
            g2[pl.ds(base2, 16)] = jnp.where(it == jstar - base2, ng2, v2)

            _rmw_lane(lk, r, m)
            _rmw_lane(li, r, wid * _CHUNK + lidx)
            return 0
        lax.fori_loop(0, _TOPK, p2, 0)

        for t in range(_TOPK // 16, 8):
            padmask = (t * 16 + it) >= _TOPK
            lk[pl.ds(t * 16, 16)] = jnp.where(padmask, jnp.int32(_MIN),
                                              lk[pl.ds(t * 16, 16)])
            li[pl.ds(t * 16, 16)] = jnp.where(padmask, jnp.int32(0),
                                              li[pl.ds(t * 16, 16)])
        pltpu.sync_copy(lk, shk.at[pl.ds(wid * 128, 128)])
        pltpu.sync_copy(li, shi.at[pl.ds(wid * 128, 128)])

    plsc.subcore_barrier()

    @pl.when(w0)
    def _():
        pltpu.sync_copy(shk, mk)
        pltpu.sync_copy(shi, mi)
        heads0 = plsc.load_gather(mk, [it * 128])
        ptrs0 = jnp.zeros((16,), jnp.int32)

        def p4(i, carry):
            heads, ptrs = carry
            m = jnp.max(heads)
            wstar = jnp.min(jnp.where(heads == m, it, jnp.int32(15)))
            p = jnp.minimum(
                jnp.min(jnp.where(it == wstar, ptrs, jnp.int32(126))),
                jnp.int32(126))
            gidx = mi[pl.ds(wstar * 128 + p, 16)][0]
            nxt = mk[pl.ds(wstar * 128 + p + 1, 16)][0]
            heads = jnp.where(it == wstar, nxt, heads)
            ptrs = jnp.where(it == wstar, p + 1, ptrs)
            _rmw_lane(selk, i, m)
            _rmw_lane(seli, i, gidx)
            return heads, ptrs
        lax.fori_loop(0, _TOPK, p4, (heads0, ptrs0))

        for t in range(8):
            sl = pl.ds(t * 16, 16)
            iv = seli[sl]
            idsb[sl] = lax.shift_right_arithmetic(iv, 14)
            k = selk[sl]
            bits = jnp.where(k >= 0, k, k ^ jnp.int32(0x7FFFFFFF))
            scb[sl] = plsc.bitcast(bits, jnp.float32)

        pltpu.sync_copy(idsb, ids_hbm)
        pltpu.sync_copy(seli, pix_hbm)
        pltpu.sync_copy(scb, sc_hbm)


def _sc_topk(keys_flat):
    mesh = plsc.VectorSubcoreMesh(core_axis_name="c", subcore_axis_name="s")
    f = pl.kernel(
        _sc_body,
        out_type=[jax.ShapeDtypeStruct((128,), jnp.int32),
                  jax.ShapeDtypeStruct((128,), jnp.int32),
                  jax.ShapeDtypeStruct((128,), jnp.float32)],
        mesh=mesh,
        compiler_params=pltpu.CompilerParams(needs_layout_passes=False),
        scratch_types=[
            pltpu.VMEM((_CHUNK,), jnp.int32),
            pltpu.VMEM((_CHUNK,), jnp.int32),
            pltpu.VMEM((_NG,), jnp.int32),
            pltpu.VMEM((32,), jnp.int32),
            pltpu.VMEM((128,), jnp.int32),
            pltpu.VMEM((128,), jnp.int32),
            pltpu.VMEM_SHARED((2048,), jnp.int32),
            pltpu.VMEM_SHARED((2048,), jnp.int32),
            pltpu.VMEM((2048,), jnp.int32),
            pltpu.VMEM((2048,), jnp.int32),
            pltpu.VMEM((128,), jnp.int32),
            pltpu.VMEM((128,), jnp.int32),
            pltpu.VMEM((128,), jnp.int32),
            pltpu.VMEM((128,), jnp.float32),
        ],
    )
    return f(keys_flat)


def _boxes_kernel(pix_ref, pwh_ref, pxy_ref, boxes_ref):
    lane1 = lax.broadcasted_iota(jnp.int32, (1, 128), 1)
    pix = pix_ref[...]
    zf = jnp.zeros((1, 128), jnp.float32)

    def body(i, carry):
        bl, bt, br, bb = carry
        flat = jnp.sum(jnp.where(lane1 == i, pix, 0))
        b = flat // (_H * _W)
        rem = flat - b * (_H * _W)
        y = rem // _W
        xcol = rem - y * _W
        sel = lane1 == xcol
        rw = pwh_ref[pl.ds(b * 2 * _H + y, 1), :]
        rh = pwh_ref[pl.ds((b * 2 + 1) * _H + y, 1), :]
        rx = pxy_ref[pl.ds(b * 2 * _H + y, 1), :]
        ry = pxy_ref[pl.ds((b * 2 + 1) * _H + y, 1), :]
        w = jnp.sum(jnp.where(sel, rw, zf))
        h = jnp.sum(jnp.where(sel, rh, zf))
        ox = jnp.sum(jnp.where(sel, rx, zf))
        oy = jnp.sum(jnp.where(sel, ry, zf))
        xc = (ox + xcol.astype(jnp.float32)) / jnp.float32(_W)
        yc = (oy + y.astype(jnp.float32)) / jnp.float32(_H)
        oh = lane1 == i
        bl = jnp.where(oh, xc - w * 0.5, bl)
        bt = jnp.where(oh, yc - h * 0.5, bt)
        br = jnp.where(oh, xc + w * 0.5, br)
        bb = jnp.where(oh, yc + h * 0.5, bb)
        return bl, bt, br, bb

    bl, bt, br, bb = lax.fori_loop(0, _TOPK, body, (zf, zf, zf, zf))
    boxes_ref[...] = jnp.concatenate([bl, bt, br, bb], axis=0)


def kernel(pheatmap, pwh, pxy_offset, pkeypoint_offset):
    del pkeypoint_offset
    keys = pl.pallas_call(
        _nms_kernel,
        grid=(_B, _NCB),
        in_specs=[pl.BlockSpec((1, _CBLK, _H, _W), lambda b, c: (b, c, 0, 0))],
        out_specs=pl.BlockSpec((1, _H, _W), lambda b, c: (b, 0, 0)),
        out_shape=jax.ShapeDtypeStruct((_B, _H, _W), jnp.int32),
        scratch_shapes=[pltpu.VMEM((_H, _W), jnp.float32)],
    )(pheatmap)

    ids_p, pix_p, sc_p = _sc_topk(keys.reshape(-1))

    boxes_p = pl.pallas_call(
        _boxes_kernel,
        out_shape=jax.ShapeDtypeStruct((4, 128), jnp.float32),
    )(pix_p.reshape(1, 128), pwh.reshape(2 * _B * _H, _W),
      pxy_offset.reshape(2 * _B * _H, _W))

    ids = ids_p[:_TOPK]
    scores = sc_p[:_TOPK]
    boxes = boxes_p[:, :_TOPK].T
    return ids, boxes, scores, scores


# speedup vs baseline: 8.5776x; 1.3166x over previous
import jax
import jax.numpy as jnp
from jax import lax
from jax.experimental import pallas as pl
from jax.experimental.pallas import tpu as pltpu
from jax.experimental.pallas import tpu_sc as plsc

_THRESHOLD = 0.18
_TOPK = 100
_B, _C, _H, _W = 8, 80, 128, 128
_MIN = -2147483648
_CBLK = 80
_NCB = _C // _CBLK
_NW = 16
_CHUNK = (_B * _H * _W) // _NW
_NG = _CHUNK // 16
_NJ = _NG // 16


def _to_key(x):
    b = lax.bitcast_convert_type(x, jnp.int32)
    return jnp.where(b >= 0, b, b ^ jnp.int32(0x7FFFFFFF))


def _nms_kernel(x_ref, out_ref, acc_ref):
    c = pl.program_id(1)
    x = x_ref[0]
    lane = lax.broadcasted_iota(jnp.int32, (_CBLK, _H, _W), 2)
    sub = lax.broadcasted_iota(jnp.int32, (_CBLK, _H, _W), 1)
    ninf = jnp.float32(-jnp.inf)
    sr = jnp.where(lane == 0, ninf, pltpu.roll(x, 1, 2))
    sl = jnp.where(lane == _W - 1, ninf, pltpu.roll(x, _W - 1, 2))
    m1 = jnp.maximum(jnp.maximum(x, sr), sl)
    su = jnp.where(sub == 0, ninf, pltpu.roll(m1, 1, 1))
    sd = jnp.where(sub == _H - 1, ninf, pltpu.roll(m1, _H - 1, 1))
    m2 = jnp.maximum(jnp.maximum(m1, su), sd)
    cand = jnp.max(jnp.where(m2 == x, x, jnp.float32(0.0)), axis=0)

    @pl.when(c == 0)
    def _():
        acc_ref[...] = cand

    @pl.when(c > 0)
    def _():
        acc_ref[...] = jnp.maximum(acc_ref[...], cand)

    @pl.when(c == _NCB - 1)
    def _():
        acc = jnp.maximum(acc_ref[...], cand)
        masked = jnp.where(acc > _THRESHOLD, acc, ninf)
        out_ref[0] = _to_key(masked)


def _iota16():
    return lax.iota(jnp.int32, 16)


def _rmw_lane(ref, pos, val):
    base = (pos // 16) * 16
    lane = pos - base
    v = ref[pl.ds(base, 16)]
    ref[pl.ds(base, 16)] = jnp.where(_iota16() == lane, val, v)


def _sc_body(keys_hbm, ids_hbm, pix_hbm, sc_hbm,
             kv, cmx, gmax, g2, lk, li, shk, shi, mk, mi,
             selk, seli, idsb, scb):
    cid = lax.axis_index("c")
    wid = lax.axis_index("s")
    active = cid == 0
    w0 = jnp.logical_and(active, wid == 0)
    it = _iota16()

    @pl.when(active)
    def _():
        pltpu.sync_copy(keys_hbm.at[pl.ds(wid * _CHUNK, _CHUNK)], kv)
        def p1(g, _):
            cmx[pl.ds(g * 16, 16)] = plsc.cummax(kv[pl.ds(g * 16, 16)])
            return 0
        lax.fori_loop(0, _NG, p1, 0)

        def p1b(j, _):
            gmax[pl.ds(j * 16, 16)] = plsc.load_gather(
                cmx, [j * 256 + it * 16 + 15])
            return 0
        lax.fori_loop(0, _NJ, p1b, 0)

        def p1c(j, _):
            cmx[pl.ds(j * 16, 16)] = plsc.cummax(gmax[pl.ds(j * 16, 16)])
            return 0
        lax.fori_loop(0, _NJ, p1c, 0)
        g2[pl.ds(0, 16)] = plsc.load_gather(cmx, [it * 16 + 15])
        g2[pl.ds(16, 16)] = plsc.load_gather(cmx, [256 + it * 16 + 15])

        def p2(r, _):
            va = g2[pl.ds(0, 16)]
            vb = g2[pl.ds(16, 16)]
            take = vb > va
            cv = jnp.where(take, vb, va)
            cj = jnp.where(take, it + 16, it)
            m = jnp.max(cv)
            jstar = jnp.min(jnp.where(cv == m, cj, jnp.int32(31)))
            gv = gmax[pl.ds(jstar * 16, 16)]
            gin = jnp.min(jnp.where(gv == m, it, jnp.int32(15)))
            gstar = jstar * 16 + gin
            kvv = kv[pl.ds(gstar * 16, 16)]
            lstar = jnp.min(jnp.where(kvv == m, it, jnp.int32(15)))
            lidx = gstar * 16 + lstar

            kvv2 = jnp.where(it == lstar, jnp.int32(_MIN), kvv)
            kv[pl.ds(gstar * 16, 16)] = kvv2
            gv2 = jnp.where(it == gin, jnp.max(kvv2), gv)
            gmax[pl.ds(jstar * 16, 16)] = gv2
            ng2 = jnp.max(gv2)
            base2 = (jstar // 16) * 16
            v2 = g2[pl.ds(base2, 16)]
            g2[pl.ds(base2, 16)] = jnp.where(it == jstar - base2, ng2, v2)

            _rmw_lane(lk, r, m)
            _rmw_lane(li, r, wid * _CHUNK + lidx)
            return 0
        lax.fori_loop(0, _TOPK, p2, 0)

        for t in range(_TOPK // 16, 8):
            padmask = (t * 16 + it) >= _TOPK
            lk[pl.ds(t * 16, 16)] = jnp.where(padmask, jnp.int32(_MIN),
                                              lk[pl.ds(t * 16, 16)])
            li[pl.ds(t * 16, 16)] = jnp.where(padmask, jnp.int32(0),
                                              li[pl.ds(t * 16, 16)])
        pltpu.sync_copy(lk, shk.at[pl.ds(wid * 128, 128)])
        pltpu.sync_copy(li, shi.at[pl.ds(wid * 128, 128)])

    plsc.subcore_barrier()

    @pl.when(w0)
    def _():
        pltpu.sync_copy(shk, mk)
        pltpu.sync_copy(shi, mi)
        heads0 = plsc.load_gather(mk, [it * 128])
        ptrs0 = jnp.zeros((16,), jnp.int32)

        def p4(i, carry):
            heads, ptrs = carry
            m = jnp.max(heads)
            wstar = jnp.min(jnp.where(heads == m, it, jnp.int32(15)))
            p = jnp.minimum(
                jnp.min(jnp.where(it == wstar, ptrs, jnp.int32(126))),
                jnp.int32(126))
            gidx = mi[pl.ds(wstar * 128 + p, 16)][0]
            nxt = mk[pl.ds(wstar * 128 + p + 1, 16)][0]
            heads = jnp.where(it == wstar, nxt, heads)
            ptrs = jnp.where(it == wstar, p + 1, ptrs)
            _rmw_lane(selk, i, m)
            _rmw_lane(seli, i, gidx)
            return heads, ptrs
        lax.fori_loop(0, _TOPK, p4, (heads0, ptrs0))

        for t in range(8):
            sl = pl.ds(t * 16, 16)
            iv = seli[sl]
            idsb[sl] = lax.shift_right_arithmetic(iv, 14)
            k = selk[sl]
            bits = jnp.where(k >= 0, k, k ^ jnp.int32(0x7FFFFFFF))
            scb[sl] = plsc.bitcast(bits, jnp.float32)

        pltpu.sync_copy(idsb, ids_hbm)
        pltpu.sync_copy(seli, pix_hbm)
        pltpu.sync_copy(scb, sc_hbm)


def _sc_topk(keys_flat):
    mesh = plsc.VectorSubcoreMesh(core_axis_name="c", subcore_axis_name="s")
    f = pl.kernel(
        _sc_body,
        out_type=[jax.ShapeDtypeStruct((128,), jnp.int32),
                  jax.ShapeDtypeStruct((128,), jnp.int32),
                  jax.ShapeDtypeStruct((128,), jnp.float32)],
        mesh=mesh,
        compiler_params=pltpu.CompilerParams(needs_layout_passes=False),
        scratch_types=[
            pltpu.VMEM((_CHUNK,), jnp.int32),
            pltpu.VMEM((_CHUNK,), jnp.int32),
            pltpu.VMEM((_NG,), jnp.int32),
            pltpu.VMEM((32,), jnp.int32),
            pltpu.VMEM((128,), jnp.int32),
            pltpu.VMEM((128,), jnp.int32),
            pltpu.VMEM_SHARED((2048,), jnp.int32),
            pltpu.VMEM_SHARED((2048,), jnp.int32),
            pltpu.VMEM((2048,), jnp.int32),
            pltpu.VMEM((2048,), jnp.int32),
            pltpu.VMEM((128,), jnp.int32),
            pltpu.VMEM((128,), jnp.int32),
            pltpu.VMEM((128,), jnp.int32),
            pltpu.VMEM((128,), jnp.float32),
        ],
    )
    return f(keys_flat)


def _boxes_kernel(pix_ref, pwh_ref, pxy_ref, boxes_ref):
    pix = lax.transpose(pix_ref[...], (1, 0))
    b = pix // (_H * _W)
    rem = pix - b * (_H * _W)
    y = rem // _W
    x = rem - y * _W
    r0 = b * (2 * _H) + y
    riota = lax.broadcasted_iota(jnp.int32, (128, 2 * _B * _H), 1)
    p0 = (riota == r0).astype(jnp.float32)
    p1 = (riota == r0 + _H).astype(jnp.float32)
    gw = jnp.dot(p0, pwh_ref[...], preferred_element_type=jnp.float32)
    gh = jnp.dot(p1, pwh_ref[...], preferred_element_type=jnp.float32)
    gx = jnp.dot(p0, pxy_ref[...], preferred_element_type=jnp.float32)
    gy = jnp.dot(p1, pxy_ref[...], preferred_element_type=jnp.float32)
    xsel = lax.broadcasted_iota(jnp.int32, (128, _W), 1) == x
    zf = jnp.zeros((128, _W), jnp.float32)
    w = jnp.sum(jnp.where(xsel, gw, zf), axis=1, keepdims=True)
    h = jnp.sum(jnp.where(xsel, gh, zf), axis=1, keepdims=True)
    ox = jnp.sum(jnp.where(xsel, gx, zf), axis=1, keepdims=True)
    oy = jnp.sum(jnp.where(xsel, gy, zf), axis=1, keepdims=True)
    xc = (ox + x.astype(jnp.float32)) / jnp.float32(_W)
    yc = (oy + y.astype(jnp.float32)) / jnp.float32(_H)
    boxes_ref[...] = jnp.concatenate(
        [xc - w * 0.5, yc - h * 0.5, xc + w * 0.5, yc + h * 0.5], axis=1)


def kernel(pheatmap, pwh, pxy_offset, pkeypoint_offset):
    del pkeypoint_offset
    keys = pl.pallas_call(
        _nms_kernel,
        grid=(_B, _NCB),
        in_specs=[pl.BlockSpec((1, _CBLK, _H, _W), lambda b, c: (b, c, 0, 0))],
        out_specs=pl.BlockSpec((1, _H, _W), lambda b, c: (b, 0, 0)),
        out_shape=jax.ShapeDtypeStruct((_B, _H, _W), jnp.int32),
        scratch_shapes=[pltpu.VMEM((_H, _W), jnp.float32)],
    )(pheatmap)

    ids_p, pix_p, sc_p = _sc_topk(keys.reshape(-1))

    boxes_p = pl.pallas_call(
        _boxes_kernel,
        out_shape=jax.ShapeDtypeStruct((128, 4), jnp.float32),
    )(pix_p.reshape(1, 128), pwh.reshape(2 * _B * _H, _W),
      pxy_offset.reshape(2 * _B * _H, _W))

    ids = ids_p[:_TOPK]
    scores = sc_p[:_TOPK]
    boxes = boxes_p[:_TOPK]
    return ids, boxes, scores, scores
